# baseline (speedup 1.0000x reference)
import functools

import jax
import jax.numpy as jnp
from jax.experimental import pallas as pl
from jax.experimental.pallas import tpu as pltpu
from jax.experimental.pallas import tpu_sc as plsc

_NUM_CORES = 2
_NUM_SUBCORES = 16
_NW = _NUM_CORES * _NUM_SUBCORES
_CHUNK = 128
_NBUF = 3


def _sc_gather2(phase_table, mag_table, idx3):
    nw, nchunk, chunk = idx3.shape
    batch = nw * nchunk * chunk
    dim = phase_table.shape[1]
    dt = phase_table.dtype
    nbuf = min(_NBUF, nchunk)
    mesh = plsc.VectorSubcoreMesh(core_axis_name="c", subcore_axis_name="s")

    @functools.partial(
        pl.kernel,
        out_type=(
            jax.ShapeDtypeStruct((batch, dim), dt),
            jax.ShapeDtypeStruct((batch, dim), dt),
        ),
        mesh=mesh,
        scratch_types=(
            [pltpu.VMEM((nchunk, chunk), jnp.int32)]
            + [pltpu.VMEM((chunk, dim), dt) for _ in range(2 * nbuf)]
            + [pltpu.SemaphoreType.DMA for _ in range(4 * nbuf + 1)]
        ),
    )
    def k(phase_hbm, mag_hbm, idx_hbm, phase_out, mag_out, idx_v, *scratch):
        pbufs = scratch[:nbuf]
        mbufs = scratch[nbuf:2 * nbuf]
        gpsems = scratch[2 * nbuf:3 * nbuf]
        gmsems = scratch[3 * nbuf:4 * nbuf]
        wpsems = scratch[4 * nbuf:5 * nbuf]
        wmsems = scratch[5 * nbuf:6 * nbuf]
        isem = scratch[6 * nbuf]
        wid = jax.lax.axis_index("s") * _NUM_CORES + jax.lax.axis_index("c")
        base = wid * (nchunk * chunk)

        pltpu.sync_copy(idx_hbm.at[wid, pl.ds(0, 1)], idx_v.at[pl.ds(0, 1)])

        def gather(j):
            s = j % nbuf
            return (
                pltpu.async_copy(phase_hbm.at[idx_v.at[j]], pbufs[s], gpsems[s]),
                pltpu.async_copy(mag_hbm.at[idx_v.at[j]], mbufs[s], gmsems[s]),
            )

        gathers, writes = {}, {}
        gathers[0] = gather(0)
        rest = pltpu.async_copy(idx_hbm.at[wid, pl.ds(1, nchunk - 1)],
                                idx_v.at[pl.ds(1, nchunk - 1)], isem)
        rest.wait()
        for j in range(1, min(nbuf - 1, nchunk)):
            gathers[j] = gather(j)
        for j in range(nchunk):
            s = j % nbuf
            nj = j + nbuf - 1
            if nj < nchunk:
                for w in writes.pop(nj - nbuf, ()):
                    w.wait()
                gathers[nj] = gather(nj)
            cp, cm = gathers.pop(j)
            out_slc = pl.ds(base + j * chunk, chunk)
            cp.wait()
            writes[j] = [pltpu.async_copy(pbufs[s], phase_out.at[out_slc],
                                          wpsems[s])]
            cm.wait()
            writes[j].append(pltpu.async_copy(mbufs[s], mag_out.at[out_slc],
                                              wmsems[s]))
        for ws in writes.values():
            for w in ws:
                w.wait()

    return k(phase_table, mag_table, idx3)


def kernel(phase_table, mag_table, indices):
    batch = indices.shape[0]
    idx3 = indices.reshape(_NW, batch // (_NW * _CHUNK), _CHUNK)
    phase, mag = _sc_gather2(phase_table, mag_table, idx3)
    return (phase, mag)


# speedup vs baseline: 1.0241x; 1.0241x over previous
import functools

import jax
import jax.numpy as jnp
from jax.experimental import pallas as pl
from jax.experimental.pallas import tpu as pltpu
from jax.experimental.pallas import tpu_sc as plsc

_NUM_CORES = 2
_NUM_SUBCORES = 16
_NW = _NUM_CORES * _NUM_SUBCORES
_CHUNK = 128
_NBUF = 7


def _sc_gather2(phase_table, mag_table, idx3):
    nw, nchunk, chunk = idx3.shape
    batch = nw * nchunk * chunk
    dim = phase_table.shape[1]
    dt = phase_table.dtype
    ntrans = 2 * nchunk
    nbuf = min(_NBUF, ntrans)
    mesh = plsc.VectorSubcoreMesh(core_axis_name="c", subcore_axis_name="s")

    @functools.partial(
        pl.kernel,
        out_type=(
            jax.ShapeDtypeStruct((batch, dim), dt),
            jax.ShapeDtypeStruct((batch, dim), dt),
        ),
        mesh=mesh,
        scratch_types=(
            [pltpu.VMEM((nchunk, chunk), jnp.int32)]
            + [pltpu.VMEM((chunk, dim), dt) for _ in range(nbuf)]
            + [pltpu.SemaphoreType.DMA for _ in range(2 * nbuf)]
        ),
    )
    def k(phase_hbm, mag_hbm, idx_hbm, phase_out, mag_out, idx_v, *scratch):
        bufs = scratch[:nbuf]
        gsems = scratch[nbuf:2 * nbuf]
        wsems = scratch[2 * nbuf:3 * nbuf]
        wid = jax.lax.axis_index("s") * _NUM_CORES + jax.lax.axis_index("c")
        pltpu.sync_copy(idx_hbm.at[wid], idx_v)
        base = wid * (nchunk * chunk)

        def src_dst(t):
            j = t // 2
            table, out = (phase_hbm, phase_out) if t % 2 == 0 else \
                         (mag_hbm, mag_out)
            return table.at[idx_v.at[j]], out.at[pl.ds(base + j * chunk, chunk)]

        def gather(t):
            s = t % nbuf
            return pltpu.async_copy(src_dst(t)[0], bufs[s], gsems[s])

        gathers, writes = {}, {}
        for t in range(min(nbuf, ntrans)):
            gathers[t] = gather(t)
        for t in range(ntrans):
            s = t % nbuf
            gathers.pop(t).wait()
            writes[t] = pltpu.async_copy(bufs[s], src_dst(t)[1], wsems[s])
            nt = t + nbuf
            if nt < ntrans:
                writes.pop(nt - nbuf).wait()
                gathers[nt] = gather(nt)
        for w in writes.values():
            w.wait()

    return k(phase_table, mag_table, idx3)


def kernel(phase_table, mag_table, indices):
    batch = indices.shape[0]
    idx3 = indices.reshape(_NW, batch // (_NW * _CHUNK), _CHUNK)
    phase, mag = _sc_gather2(phase_table, mag_table, idx3)
    return (phase, mag)
